# Initial kernel scaffold; baseline (speedup 1.0000x reference)
import functools

import jax
import jax.numpy as jnp
from jax import lax
from jax.experimental import pallas as pl
from jax.experimental.pallas import tpu as pltpu
from jax.experimental.pallas import tpu_sc as plsc

N_NODES = 10000
N_PAD = 10240
E_EDGES = 320000
E_PAD = 323584
D_IN = 128
H1 = 256
H2 = 128
H3 = 128

NUM_SC = 2
NUM_TILES = 16
STRIP = N_PAD // NUM_TILES
CHUNK = 128

_MESH = plsc.VectorSubcoreMesh(core_axis_name="c", subcore_axis_name="s")


def _deg_body(dst_hbm, deg_out, shared, part, dbuf, accv, tmp):
    c = lax.axis_index("c")
    s = lax.axis_index("s")
    ed = E_PAD // (NUM_SC * NUM_TILES)

    def zero_body(i, _):
        part[pl.ds(i * 16, 16)] = jnp.zeros((16,), jnp.float32)
        return 0

    lax.fori_loop(0, N_PAD // 16, zero_body, 0)

    base = (c * NUM_TILES + s) * ed

    def chunk_body(k, _):
        pltpu.sync_copy(dst_hbm.at[pl.ds(base + k * CHUNK, CHUNK)], dbuf)

        def inner(j, _):
            idx = dbuf[pl.ds(j * 16, 16)]
            plsc.addupdate_scatter(part, [idx], jnp.ones((16,), jnp.float32))
            return 0

        lax.fori_loop(0, CHUNK // 16, inner, 0)
        return 0

    lax.fori_loop(0, ed // CHUNK, chunk_body, 0)

    pltpu.sync_copy(part, shared.at[s])
    plsc.subcore_barrier()

    r0 = s * STRIP
    pltpu.sync_copy(shared.at[0, pl.ds(r0, STRIP)], accv)

    def red_body(k, _):
        pltpu.sync_copy(shared.at[k, pl.ds(r0, STRIP)], tmp)

        def add16(j, _):
            sl = pl.ds(j * 16, 16)
            accv[sl] = accv[sl] + tmp[sl]
            return 0

        lax.fori_loop(0, STRIP // 16, add16, 0)
        return 0

    lax.fori_loop(1, NUM_TILES, red_body, 0)
    pltpu.sync_copy(accv, deg_out.at[c, pl.ds(r0, STRIP)])


_deg_kernel = pl.kernel(
    _deg_body,
    out_type=jax.ShapeDtypeStruct((NUM_SC, N_PAD), jnp.float32),
    mesh=_MESH,
    scratch_types=[
        pltpu.VMEM_SHARED((NUM_TILES, N_PAD), jnp.float32),
        pltpu.VMEM((N_PAD,), jnp.float32),
        pltpu.VMEM((CHUNK,), jnp.int32),
        pltpu.VMEM((STRIP,), jnp.float32),
        pltpu.VMEM((STRIP,), jnp.float32),
    ],
)


def _make_scatter(fh):
    def body(h_hbm, src_hbm, dst_hbm, out_hbm, acc, isrc, idst, msg, buf, gsem):
        c = lax.axis_index("c")
        s = lax.axis_index("s")
        r0 = s * STRIP

        def init_body(i, _):
            off = r0 + i * CHUNK
            pltpu.sync_copy(h_hbm.at[c].at[pl.ds(off, CHUNK)], buf)
            pltpu.sync_copy(buf, acc.at[pl.ds(off, CHUNK)])
            return 0

        lax.fori_loop(0, STRIP // CHUNK, init_body, 0)
        plsc.subcore_barrier()

        et = E_PAD // NUM_TILES
        base = s * et

        def edge_body(k, _):
            e0 = base + k * CHUNK
            pltpu.sync_copy(src_hbm.at[pl.ds(e0, CHUNK)], isrc)
            pltpu.sync_copy(dst_hbm.at[pl.ds(e0, CHUNK)], idst)
            pltpu.async_copy(h_hbm.at[c].at[isrc], msg, gsem).wait()
            pltpu.sync_copy(msg, acc.at[idst], add=True)
            return 0

        lax.fori_loop(0, et // CHUNK, edge_body, 0)
        plsc.subcore_barrier()

        def wb_body(i, _):
            off = r0 + i * CHUNK
            pltpu.sync_copy(acc.at[pl.ds(off, CHUNK)], buf)
            pltpu.sync_copy(buf, out_hbm.at[c].at[pl.ds(off, CHUNK)])
            return 0

        lax.fori_loop(0, STRIP // CHUNK, wb_body, 0)

    return pl.kernel(
        body,
        out_type=jax.ShapeDtypeStruct((NUM_SC, N_PAD, fh), jnp.float32),
        mesh=_MESH,
        scratch_types=[
            pltpu.VMEM_SHARED((N_PAD, fh), jnp.float32),
            pltpu.VMEM((CHUNK,), jnp.int32),
            pltpu.VMEM((CHUNK,), jnp.int32),
            pltpu.VMEM((CHUNK, fh), jnp.float32),
            pltpu.VMEM((CHUNK, fh), jnp.float32),
            pltpu.SemaphoreType.DMA,
        ],
    )


_scatter_128 = _make_scatter(128)
_scatter_64 = _make_scatter(64)


ROWS = 512


def _dis_from_deg(deg_blk):
    deg = deg_blk[0] + deg_blk[1] + 1.0
    return lax.rsqrt(deg)[:, None]


def _pre1_body(x_ref, deg_ref, w_ref, out_ref):
    dis = _dis_from_deg(deg_ref[...])
    h = jnp.dot(x_ref[...], w_ref[...], preferred_element_type=jnp.float32)
    hp = h * dis
    fh = h.shape[1] // 2
    out_ref[0] = hp[:, :fh]
    out_ref[1] = hp[:, fh:]


def _ln(u, g, b):
    m = jnp.mean(u, axis=1, keepdims=True)
    d = u - m
    v = jnp.mean(d * d, axis=1, keepdims=True)
    return d * lax.rsqrt(v + 1e-5) * g + b


def _post_pre_body(t_ref, deg_ref, cb_ref, g_ref, b_ref, w_ref, out_ref):
    dis = _dis_from_deg(deg_ref[...])
    t = jnp.concatenate([t_ref[0], t_ref[1]], axis=1)
    u = jax.nn.relu(t * dis + cb_ref[...])
    u = _ln(u, g_ref[...], b_ref[...])
    h = jnp.dot(u, w_ref[...], preferred_element_type=jnp.float32) * dis
    fh = h.shape[1] // 2
    out_ref[0] = h[:, :fh]
    out_ref[1] = h[:, fh:]


def _final_body(t_ref, deg_ref, cb_ref, g_ref, b_ref, x_ref,
                w1_ref, b1_ref, w2_ref, b2_ref, w3_ref, b3_ref, w4_ref, b4_ref,
                out_ref):
    dis = _dis_from_deg(deg_ref[...])
    t = jnp.concatenate([t_ref[0], t_ref[1]], axis=1)
    u = jax.nn.relu(t * dis + cb_ref[...])
    gg = _ln(u, g_ref[...], b_ref[...])

    x = x_ref[...]
    f = jax.nn.relu(jnp.dot(x, w1_ref[...], preferred_element_type=jnp.float32) + b1_ref[...])
    f = jax.nn.relu(jnp.dot(f, w2_ref[...], preferred_element_type=jnp.float32) + b2_ref[...])
    f = jax.nn.relu(jnp.dot(f, w3_ref[...], preferred_element_type=jnp.float32) + b3_ref[...])
    f = jnp.dot(f, w4_ref[...], preferred_element_type=jnp.float32) + b4_ref[...]
    out_ref[...] = (gg + f) * 0.5


def _row_grid():
    return N_PAD // ROWS


def _spec_rows(fdim):
    return pl.BlockSpec((ROWS, fdim), lambda i: (i, 0))


def _spec_halves(fh):
    return pl.BlockSpec((NUM_SC, ROWS, fh), lambda i: (0, i, 0))


def _spec_deg():
    return pl.BlockSpec((NUM_SC, ROWS), lambda i: (0, i))


def _spec_full(shape):
    nd = len(shape)
    return pl.BlockSpec(shape, lambda i: (0,) * nd)


def _pre1(x_pad, deg2, w1):
    return pl.pallas_call(
        _pre1_body,
        grid=(_row_grid(),),
        in_specs=[_spec_rows(D_IN), _spec_deg(), _spec_full(w1.shape)],
        out_specs=_spec_halves(H1 // 2),
        out_shape=jax.ShapeDtypeStruct((NUM_SC, N_PAD, H1 // 2), jnp.float32),
    )(x_pad, deg2, w1)


def _post_pre(t, deg2, cb, g, b, w, fin, fout):
    return pl.pallas_call(
        _post_pre_body,
        grid=(_row_grid(),),
        in_specs=[
            _spec_halves(fin // 2), _spec_deg(), _spec_full((1, fin)),
            _spec_full((1, fin)), _spec_full((1, fin)), _spec_full(w.shape),
        ],
        out_specs=_spec_halves(fout // 2),
        out_shape=jax.ShapeDtypeStruct((NUM_SC, N_PAD, fout // 2), jnp.float32),
    )(t, deg2, cb, g, b, w)


def _final(t, deg2, cb, g, b, x_pad, w1, b1, w2, b2, w3, b3, w4, b4):
    return pl.pallas_call(
        _final_body,
        grid=(_row_grid(),),
        in_specs=[
            _spec_halves(H3 // 2), _spec_deg(), _spec_full((1, H3)),
            _spec_full((1, H3)), _spec_full((1, H3)), _spec_rows(D_IN),
            _spec_full(w1.shape), _spec_full((1, H1)),
            _spec_full(w2.shape), _spec_full((1, H1)),
            _spec_full(w3.shape), _spec_full((1, H2)),
            _spec_full(w4.shape), _spec_full((1, H3)),
        ],
        out_specs=_spec_rows(H3),
        out_shape=jax.ShapeDtypeStruct((N_PAD, H3), jnp.float32),
    )(t, deg2, cb, g, b, x_pad, w1, b1, w2, b2, w3, b3, w4, b4)


def kernel(x, edge_index, conv1_W, conv1_b, conv2_W, conv2_b, conv3_W, conv3_b,
           norm1_g, norm1_b, norm2_g, norm2_b, norm3_g, norm3_b,
           fc1_W, fc1_b, fc2_W, fc2_b, fc3_W, fc3_b, fc4_W, fc4_b):
    x_pad = jnp.pad(x, ((0, N_PAD - N_NODES), (0, 0)))
    src = jnp.pad(edge_index[0], (0, E_PAD - E_EDGES), constant_values=N_NODES)
    dst = jnp.pad(edge_index[1], (0, E_PAD - E_EDGES), constant_values=N_NODES)

    r = lambda v: v[None, :]

    deg2 = _deg_kernel(dst)

    h1 = _pre1(x_pad, deg2, conv1_W)
    t1 = _scatter_128(h1, src, dst)
    h2 = _post_pre(t1, deg2, r(conv1_b), r(norm1_g), r(norm1_b), conv2_W, H1, H2)
    t2 = _scatter_64(h2, src, dst)
    h3 = _post_pre(t2, deg2, r(conv2_b), r(norm2_g), r(norm2_b), conv3_W, H2, H3)
    t3 = _scatter_64(h3, src, dst)
    out = _final(t3, deg2, r(conv3_b), r(norm3_g), r(norm3_b), x_pad,
                 fc1_W, r(fc1_b), fc2_W, r(fc2_b), fc3_W, r(fc3_b),
                 fc4_W, r(fc4_b))
    return out[:N_NODES]


# speedup vs baseline: 7.4523x; 7.4523x over previous
import functools

import jax
import jax.numpy as jnp
from jax import lax
from jax.experimental import pallas as pl
from jax.experimental.pallas import tpu as pltpu
from jax.experimental.pallas import tpu_sc as plsc

N_NODES = 10000
N_PAD = 10240
E_EDGES = 320000
E_PAD = 323584
D_IN = 128
H1 = 256
H2 = 128
H3 = 128

NUM_SC = 2
NUM_TILES = 16
STRIP = N_PAD // NUM_TILES
CHUNK = 128

_MESH = plsc.VectorSubcoreMesh(core_axis_name="c", subcore_axis_name="s")


DEGW = 128


def _deg_body(dst_hbm, zeros_hbm, ones_hbm, deg_out, dacc, ones_v, idst, buf):
    c = lax.axis_index("c")
    s = lax.axis_index("s")
    ed = E_PAD // (NUM_SC * NUM_TILES)
    r0 = s * STRIP

    pltpu.sync_copy(ones_hbm, ones_v)

    def init_body(i, _):
        off = r0 + i * CHUNK
        pltpu.sync_copy(zeros_hbm.at[pl.ds(0, CHUNK)], buf)
        pltpu.sync_copy(buf, dacc.at[pl.ds(off, CHUNK)])
        return 0

    lax.fori_loop(0, STRIP // CHUNK, init_body, 0)
    plsc.subcore_barrier()

    base = (c * NUM_TILES + s) * ed

    def chunk_body(k, _):
        pltpu.sync_copy(dst_hbm.at[pl.ds(base + k * CHUNK, CHUNK)], idst)
        pltpu.sync_copy(ones_v, dacc.at[idst], add=True)
        return 0

    lax.fori_loop(0, ed // CHUNK, chunk_body, 0)
    plsc.subcore_barrier()

    def wb_body(i, _):
        off = r0 + i * CHUNK
        pltpu.sync_copy(dacc.at[pl.ds(off, CHUNK)], buf)
        pltpu.sync_copy(buf, deg_out.at[c].at[pl.ds(off, CHUNK)])
        return 0

    lax.fori_loop(0, STRIP // CHUNK, wb_body, 0)


_deg_kernel = pl.kernel(
    _deg_body,
    out_type=jax.ShapeDtypeStruct((NUM_SC, N_PAD, DEGW), jnp.float32),
    mesh=_MESH,
    scratch_types=[
        pltpu.VMEM_SHARED((N_PAD, DEGW), jnp.float32),
        pltpu.VMEM((CHUNK, DEGW), jnp.float32),
        pltpu.VMEM((CHUNK,), jnp.int32),
        pltpu.VMEM((CHUNK, DEGW), jnp.float32),
    ],
)


def _make_scatter(fh, feat_split):
    def body(h_hbm, src_hbm, dst_hbm, zero_hbm, out_hbm,
             acc, isrc, idst, msg, buf, gsem):
        c = lax.axis_index("c")
        s = lax.axis_index("s")
        r0 = s * STRIP

        def table():
            return h_hbm.at[c] if feat_split else h_hbm

        def init_body(i, _):
            off = r0 + i * CHUNK
            if feat_split:
                pltpu.sync_copy(table().at[pl.ds(off, CHUNK)], buf)
            else:
                @pl.when(c == 0)
                def _():
                    pltpu.sync_copy(table().at[pl.ds(off, CHUNK)], buf)

                @pl.when(c != 0)
                def _():
                    pltpu.sync_copy(zero_hbm, buf)
            pltpu.sync_copy(buf, acc.at[pl.ds(off, CHUNK)])
            return 0

        lax.fori_loop(0, STRIP // CHUNK, init_body, 0)
        plsc.subcore_barrier()

        if feat_split:
            et = E_PAD // NUM_TILES
            base = s * et
        else:
            et = E_PAD // (NUM_SC * NUM_TILES)
            base = (c * NUM_TILES + s) * et

        def edge_body(k, _):
            e0 = base + k * CHUNK
            pltpu.sync_copy(src_hbm.at[pl.ds(e0, CHUNK)], isrc)
            pltpu.sync_copy(dst_hbm.at[pl.ds(e0, CHUNK)], idst)
            pltpu.async_copy(table().at[isrc], msg, gsem).wait()
            pltpu.sync_copy(msg, acc.at[idst], add=True)
            return 0

        lax.fori_loop(0, et // CHUNK, edge_body, 0)
        plsc.subcore_barrier()

        def wb_body(i, _):
            off = r0 + i * CHUNK
            pltpu.sync_copy(acc.at[pl.ds(off, CHUNK)], buf)
            pltpu.sync_copy(buf, out_hbm.at[c].at[pl.ds(off, CHUNK)])
            return 0

        lax.fori_loop(0, STRIP // CHUNK, wb_body, 0)

    return pl.kernel(
        body,
        out_type=jax.ShapeDtypeStruct((NUM_SC, N_PAD, fh), jnp.float32),
        mesh=_MESH,
        scratch_types=[
            pltpu.VMEM_SHARED((N_PAD, fh), jnp.float32),
            pltpu.VMEM((CHUNK,), jnp.int32),
            pltpu.VMEM((CHUNK,), jnp.int32),
            pltpu.VMEM((CHUNK, fh), jnp.float32),
            pltpu.VMEM((CHUNK, fh), jnp.float32),
            pltpu.SemaphoreType.DMA,
        ],
    )


_scatter_feat = _make_scatter(128, True)
_scatter_edge = _make_scatter(128, False)


ROWS = 512


def _dis_from_deg(deg_blk):
    deg = deg_blk[0, :, 0:1] + deg_blk[1, :, 0:1] + 1.0
    return lax.rsqrt(deg)


def _pre1_body(x_ref, deg_ref, w_ref, out_ref):
    dis = _dis_from_deg(deg_ref[...])
    h = jnp.dot(x_ref[...], w_ref[...], preferred_element_type=jnp.float32)
    hp = h * dis
    fh = h.shape[1] // 2
    out_ref[0] = hp[:, :fh]
    out_ref[1] = hp[:, fh:]


def _ln(u, g, b):
    m = jnp.mean(u, axis=1, keepdims=True)
    d = u - m
    v = jnp.mean(d * d, axis=1, keepdims=True)
    return d * lax.rsqrt(v + 1e-5) * g + b


def _combine_t(t_blk, concat):
    if concat:
        return jnp.concatenate([t_blk[0], t_blk[1]], axis=1)
    return t_blk[0] + t_blk[1]


def _make_post_pre_body(concat):
    def body(t_ref, deg_ref, cb_ref, g_ref, b_ref, w_ref, out_ref):
        dis = _dis_from_deg(deg_ref[...])
        t = _combine_t(t_ref[...], concat)
        u = jax.nn.relu(t * dis + cb_ref[...])
        u = _ln(u, g_ref[...], b_ref[...])
        out_ref[...] = jnp.dot(u, w_ref[...],
                               preferred_element_type=jnp.float32) * dis

    return body


def _final_body(t_ref, deg_ref, cb_ref, g_ref, b_ref, x_ref,
                w1_ref, b1_ref, w2_ref, b2_ref, w3_ref, b3_ref, w4_ref, b4_ref,
                out_ref):
    dis = _dis_from_deg(deg_ref[...])
    t = _combine_t(t_ref[...], False)
    u = jax.nn.relu(t * dis + cb_ref[...])
    gg = _ln(u, g_ref[...], b_ref[...])

    x = x_ref[...]
    f = jax.nn.relu(jnp.dot(x, w1_ref[...], preferred_element_type=jnp.float32) + b1_ref[...])
    f = jax.nn.relu(jnp.dot(f, w2_ref[...], preferred_element_type=jnp.float32) + b2_ref[...])
    f = jax.nn.relu(jnp.dot(f, w3_ref[...], preferred_element_type=jnp.float32) + b3_ref[...])
    f = jnp.dot(f, w4_ref[...], preferred_element_type=jnp.float32) + b4_ref[...]
    out_ref[...] = (gg + f) * 0.5


def _row_grid():
    return N_PAD // ROWS


def _spec_rows(fdim):
    return pl.BlockSpec((ROWS, fdim), lambda i: (i, 0))


def _spec_halves(fh):
    return pl.BlockSpec((NUM_SC, ROWS, fh), lambda i: (0, i, 0))


def _spec_deg():
    return pl.BlockSpec((NUM_SC, ROWS, DEGW), lambda i: (0, i, 0))


def _spec_full(shape):
    nd = len(shape)
    return pl.BlockSpec(shape, lambda i: (0,) * nd)


def _pre1(x_pad, deg2, w1):
    return pl.pallas_call(
        _pre1_body,
        grid=(_row_grid(),),
        in_specs=[_spec_rows(D_IN), _spec_deg(), _spec_full(w1.shape)],
        out_specs=_spec_halves(H1 // 2),
        out_shape=jax.ShapeDtypeStruct((NUM_SC, N_PAD, H1 // 2), jnp.float32),
    )(x_pad, deg2, w1)


def _post_pre(t, deg2, cb, g, b, w, fin, fout, concat):
    thw = fin // 2 if concat else fin
    return pl.pallas_call(
        _make_post_pre_body(concat),
        grid=(_row_grid(),),
        in_specs=[
            _spec_halves(thw), _spec_deg(), _spec_full((1, fin)),
            _spec_full((1, fin)), _spec_full((1, fin)), _spec_full(w.shape),
        ],
        out_specs=_spec_rows(fout),
        out_shape=jax.ShapeDtypeStruct((N_PAD, fout), jnp.float32),
    )(t, deg2, cb, g, b, w)


def _final(t, deg2, cb, g, b, x_pad, w1, b1, w2, b2, w3, b3, w4, b4):
    return pl.pallas_call(
        _final_body,
        grid=(_row_grid(),),
        in_specs=[
            _spec_halves(H3), _spec_deg(), _spec_full((1, H3)),
            _spec_full((1, H3)), _spec_full((1, H3)), _spec_rows(D_IN),
            _spec_full(w1.shape), _spec_full((1, H1)),
            _spec_full(w2.shape), _spec_full((1, H1)),
            _spec_full(w3.shape), _spec_full((1, H2)),
            _spec_full(w4.shape), _spec_full((1, H3)),
        ],
        out_specs=_spec_rows(H3),
        out_shape=jax.ShapeDtypeStruct((N_PAD, H3), jnp.float32),
    )(t, deg2, cb, g, b, x_pad, w1, b1, w2, b2, w3, b3, w4, b4)


def kernel(x, edge_index, conv1_W, conv1_b, conv2_W, conv2_b, conv3_W, conv3_b,
           norm1_g, norm1_b, norm2_g, norm2_b, norm3_g, norm3_b,
           fc1_W, fc1_b, fc2_W, fc2_b, fc3_W, fc3_b, fc4_W, fc4_b):
    x_pad = jnp.pad(x, ((0, N_PAD - N_NODES), (0, 0)))
    src = jnp.pad(edge_index[0], (0, E_PAD - E_EDGES), constant_values=N_NODES)
    dst = jnp.pad(edge_index[1], (0, E_PAD - E_EDGES), constant_values=N_NODES)

    r = lambda v: v[None, :]

    deg2 = _deg_kernel(dst, jnp.zeros((CHUNK, DEGW), jnp.float32),
                       jnp.ones((CHUNK, DEGW), jnp.float32))

    zc = jnp.zeros((CHUNK, 128), jnp.float32)
    h1 = _pre1(x_pad, deg2, conv1_W)
    t1 = _scatter_feat(h1, src, dst, zc)
    h2 = _post_pre(t1, deg2, r(conv1_b), r(norm1_g), r(norm1_b), conv2_W,
                   H1, H2, concat=True)
    t2 = _scatter_edge(h2, src, dst, zc)
    h3 = _post_pre(t2, deg2, r(conv2_b), r(norm2_g), r(norm2_b), conv3_W,
                   H2, H3, concat=False)
    t3 = _scatter_edge(h3, src, dst, zc)
    out = _final(t3, deg2, r(conv3_b), r(norm3_g), r(norm3_b), x_pad,
                 fc1_W, r(fc1_b), fc2_W, r(fc2_b), fc3_W, r(fc3_b),
                 fc4_W, r(fc4_b))
    return out[:N_NODES]
